# baseline (speedup 1.0000x reference)
import functools

import jax
import jax.numpy as jnp
from jax import lax
from jax.experimental import pallas as pl
from jax.experimental.pallas import tpu as pltpu
from jax.experimental.pallas import tpu_sc as plsc

H = 2048
E = 16
N = 16384
TOPK = 2
BT = 512
NW = 32
C = N // NW
L = 16


def _fused_body(x_ref, w_ref, rw_ref, se_ref):
    logits = lax.dot_general(
        x_ref[...], w_ref[...],
        dimension_numbers=(((1,), (1,)), ((), ())),
        preferred_element_type=jnp.float32,
    )
    neg_inf = jnp.float32(-jnp.inf)
    eiota = lax.broadcasted_iota(jnp.int32, (BT, E), 1)
    m1 = jnp.max(logits, axis=1, keepdims=True)
    idx1 = jnp.min(jnp.where(logits == m1, eiota, jnp.int32(E)),
                   axis=1, keepdims=True)
    masked = jnp.where(eiota == idx1, neg_inf, logits)
    m2 = jnp.max(masked, axis=1, keepdims=True)
    idx2 = jnp.min(jnp.where(masked == m2, eiota, jnp.int32(E)),
                   axis=1, keepdims=True)
    ex = jnp.exp(m2 - m1)
    denom = jnp.float32(1.0) + ex
    w0 = jnp.float32(1.0) / denom
    w1 = ex / denom
    rw_ref[...] = jnp.concatenate([w0, w1], axis=1)
    se_ref[...] = jnp.concatenate([idx1, idx2], axis=1)


def _fused_tc(x, w):
    return pl.pallas_call(
        _fused_body,
        grid=(N // BT,),
        in_specs=[
            pl.BlockSpec((BT, H), lambda i: (i, 0)),
            pl.BlockSpec((E, H), lambda i: (0, 0)),
        ],
        out_specs=[
            pl.BlockSpec((BT, TOPK), lambda i: (i, 0)),
            pl.BlockSpec((BT, TOPK), lambda i: (i, 0)),
        ],
        out_shape=[
            jax.ShapeDtypeStruct((N, TOPK), jnp.float32),
            jax.ShapeDtypeStruct((N, TOPK), jnp.int32),
        ],
    )(x, w)


def kernel(hidden_states, weight):
    routing_weights, selected_experts = _fused_tc(hidden_states, weight)
    return routing_weights, selected_experts


# speedup vs baseline: 1.2185x; 1.2185x over previous
import functools

import jax
import jax.numpy as jnp
from jax import lax
from jax.experimental import pallas as pl
from jax.experimental.pallas import tpu as pltpu
from jax.experimental.pallas import tpu_sc as plsc

H = 2048
E = 16
N = 16384
TOPK = 2
BT = 2048
NW = 32
C = N // NW
L = 16


def _fused_body(x_ref, w_ref, rw_ref, se_ref):
    logits = lax.dot_general(
        x_ref[...], w_ref[...],
        dimension_numbers=(((1,), (1,)), ((), ())),
        preferred_element_type=jnp.float32,
    )
    neg_inf = jnp.float32(-jnp.inf)
    eiota = lax.broadcasted_iota(jnp.int32, (BT, E), 1)
    m1 = jnp.max(logits, axis=1, keepdims=True)
    idx1 = jnp.min(jnp.where(logits == m1, eiota, jnp.int32(E)),
                   axis=1, keepdims=True)
    masked = jnp.where(eiota == idx1, neg_inf, logits)
    m2 = jnp.max(masked, axis=1, keepdims=True)
    idx2 = jnp.min(jnp.where(masked == m2, eiota, jnp.int32(E)),
                   axis=1, keepdims=True)
    ex = jnp.exp(m2 - m1)
    denom = jnp.float32(1.0) + ex
    w0 = jnp.float32(1.0) / denom
    w1 = ex / denom
    rw_ref[...] = jnp.concatenate([w0, w1], axis=1)
    se_ref[...] = jnp.concatenate([idx1, idx2], axis=1)


def _fused_tc(x, w):
    return pl.pallas_call(
        _fused_body,
        grid=(N // BT,),
        in_specs=[
            pl.BlockSpec((BT, H), lambda i: (i, 0)),
            pl.BlockSpec((E, H), lambda i: (0, 0)),
        ],
        out_specs=[
            pl.BlockSpec((BT, TOPK), lambda i: (i, 0)),
            pl.BlockSpec((BT, TOPK), lambda i: (i, 0)),
        ],
        out_shape=[
            jax.ShapeDtypeStruct((N, TOPK), jnp.float32),
            jax.ShapeDtypeStruct((N, TOPK), jnp.int32),
        ],
    )(x, w)


def kernel(hidden_states, weight):
    routing_weights, selected_experts = _fused_tc(hidden_states, weight)
    return routing_weights, selected_experts
